# baseline (speedup 1.0000x reference)
import functools

import jax
import jax.numpy as jnp
from jax import lax
from jax.experimental import pallas as pl
from jax.experimental.pallas import tpu as pltpu
from jax.experimental.pallas import tpu_sc as plsc

N = 10000
E = 320000
D = 128
NPAD = 10240
NTILES = 32
PER_TILE = E // NTILES
CH = 40
NCH = PER_TILE // CH
RPT = NPAD // 16
UNROLL = 6
STEADY = (NCH - 4) // UNROLL


def _make_sc_agg():
  mesh = plsc.VectorSubcoreMesh(core_axis_name="c", subcore_axis_name="s")

  @functools.partial(
      pl.kernel,
      mesh=mesh,
      out_type=jax.ShapeDtypeStruct((2 * NPAD, D), jnp.float32),
      scratch_types=[
          pltpu.VMEM((CH, D), jnp.float32),
          pltpu.VMEM((CH, 2 * D), jnp.float32),
          pltpu.VMEM((CH, D), jnp.float32),
          pltpu.VMEM((CH, 2 * D), jnp.float32),
          pltpu.VMEM((CH, D), jnp.float32),
          pltpu.VMEM((CH, D), jnp.float32),
          pltpu.VMEM((CH,), jnp.int32),
          pltpu.VMEM((CH,), jnp.int32),
          pltpu.VMEM((CH,), jnp.int32),
          pltpu.VMEM((CH,), jnp.int32),
          pltpu.VMEM((CH,), jnp.int32),
          pltpu.VMEM((CH,), jnp.int32),
          pltpu.VMEM((CH,), jnp.int32),
          pltpu.VMEM((CH,), jnp.int32),
          pltpu.VMEM_SHARED((NPAD, D), jnp.float32),
          pltpu.SemaphoreType.DMA,
          pltpu.SemaphoreType.DMA,
          pltpu.SemaphoreType.DMA,
          pltpu.SemaphoreType.DMA,
          pltpu.SemaphoreType.DMA,
          pltpu.SemaphoreType.DMA,
          pltpu.SemaphoreType.DMA,
          pltpu.SemaphoreType.DMA,
          pltpu.SemaphoreType.DMA,
      ],
  )
  def sc_agg(k_hbm, qv_hbm, src_hbm, dst_hbm, zeros_hbm, out_hbm,
             kb0, qvb0, kb1, qvb1, mb0, mb1,
             is0, id0, is1, id1, is2, id2, pd0, pd1,
             acc, semg0, semg1, semi0, semi1, semi2,
             semsc0, semsc1, sempd0, sempd1):
    c = lax.axis_index("c")
    s = lax.axis_index("s")
    tid = c * 16 + s
    G = ((kb0, qvb0, semg0), (kb1, qvb1, semg1))
    I = ((is0, id0, semi0), (is1, id1, semi1), (is2, id2, semi2))
    M = ((mb0, pd0, semsc0, sempd0), (mb1, pd1, semsc1, sempd1))

    pltpu.sync_copy(zeros_hbm, acc.at[pl.ds(s * RPT, RPT)])
    plsc.subcore_barrier()

    def fire_idx(ci, iu):
      isrc, idst, semi = I[iu]
      base = tid * PER_TILE + ci * CH
      pltpu.async_copy(src_hbm.at[pl.ds(base, CH)], isrc, semi)
      pltpu.async_copy(dst_hbm.at[pl.ds(base, CH)], idst, semi)

    def fire_gather(gu, iu):
      isrc, idst, semi = I[iu]
      kb, qvb, semg = G[gu]
      pltpu.make_async_copy(src_hbm.at[pl.ds(0, CH)], isrc, semi).wait()
      pltpu.make_async_copy(dst_hbm.at[pl.ds(0, CH)], idst, semi).wait()
      pltpu.async_copy(k_hbm.at[idst], kb, semg)
      pltpu.async_copy(qv_hbm.at[isrc], qvb, semg)

    def fire_pds(ci, mu):
      _, pd, _, sempd = M[mu]
      base = tid * PER_TILE + ci * CH
      pltpu.async_copy(dst_hbm.at[pl.ds(base, CH)], pd, sempd)

    def consume(gu, iu, mu, scwait=True):
      isrc, idst, semi = I[iu]
      kb, qvb, semg = G[gu]
      mb, pd, semsc, sempd = M[mu]
      pltpu.make_async_copy(k_hbm.at[idst], kb, semg).wait()
      pltpu.make_async_copy(qv_hbm.at[isrc], qvb, semg).wait()
      if scwait:
        pltpu.make_async_copy(mb, acc.at[pd], semsc).wait()
      pltpu.make_async_copy(dst_hbm.at[pl.ds(0, CH)], pd, sempd).wait()

      def row(r2, rc):
        for rr in range(2):
          r = r2 * 2 + rr
          for cc in range(D // 16):
            sl = pl.ds(cc * 16, 16)
            kv = kb[r, sl]
            qv = qvb[r, sl]
            vv = qvb[r, pl.ds(D + cc * 16, 16)]
            g = 1.0 + jnp.exp(-(kv + qv))
            mb[r, sl] = vv / g
        return rc

      lax.fori_loop(0, CH // 2, row, 0)
      pltpu.async_copy(mb, acc.at[pd], semsc, add=True)

    def body(ci, u, do_gather=True, do_idx=True, do_pds=True, scwait=True):
      if do_gather:
        fire_gather((u + 1) % 2, (u + 1) % 3)
      consume(u % 2, u % 3, u % 2, scwait)
      if do_pds:
        fire_pds(ci + 2, u % 2)
      if do_idx:
        fire_idx(ci + 3, u % 3)

    fire_idx(0, 0)
    fire_idx(1, 1)
    fire_idx(2, 2)
    fire_pds(0, 0)
    fire_pds(1, 1)
    fire_gather(0, 0)

    body(0, 0, scwait=False)
    body(1, 1, scwait=False)

    def steady(j, carry):
      ci = 2 + j * UNROLL
      for u in range(UNROLL):
        body(ci + u, (2 + u) % 6)
      return carry

    lax.fori_loop(0, STEADY, steady, 0)

    body(NCH - 2, (NCH - 2) % 6, do_idx=False, do_pds=False)
    body(NCH - 1, (NCH - 1) % 6, do_gather=False, do_idx=False, do_pds=False)

    mb, pd, semsc, _ = M[(NCH - 2) % 2]
    pltpu.make_async_copy(mb, acc.at[pd], semsc).wait()
    mb, pd, semsc, _ = M[(NCH - 1) % 2]
    pltpu.make_async_copy(mb, acc.at[pd], semsc).wait()
    isrc, idst, semi = I[NCH % 3]
    pltpu.make_async_copy(src_hbm.at[pl.ds(0, CH)], isrc, semi).wait()
    pltpu.make_async_copy(dst_hbm.at[pl.ds(0, CH)], idst, semi).wait()

    plsc.subcore_barrier()
    pltpu.sync_copy(acc.at[pl.ds(s * RPT, RPT)],
                    out_hbm.at[pl.ds(c * NPAD + s * RPT, RPT)])

  return sc_agg


_SC_AGG_CACHE = []


def _sc_agg(*args):
  if not _SC_AGG_CACHE:
    _SC_AGG_CACHE.append(_make_sc_agg())
  return _SC_AGG_CACHE[0](*args)


RB = 640
GRID = NPAD // RB

_row_spec = pl.BlockSpec((RB, D), lambda i: (i, 0))
_agg0_spec = pl.BlockSpec((RB, D), lambda i: (i, 0))
_agg1_spec = pl.BlockSpec((RB, D), lambda i: (GRID + i, 0))
_w_spec = pl.BlockSpec((4 * D, D), lambda i: (0, 0))
_b_spec = pl.BlockSpec((4, D), lambda i: (0, 0))
_wh_spec = pl.BlockSpec((D, D), lambda i: (0, 0))
_bh_spec = pl.BlockSpec((1, D), lambda i: (0, 0))

_DN = (((1,), (1,)), ((), ()))


def _xwt(x, w):
  return lax.dot_general(x, w, _DN, preferred_element_type=jnp.float32)


def _tc1_body(x_ref, wh, beff, w4, b4, k_o, qv_o, b_o):
  x0 = _xwt(x_ref[...], wh[...]) + beff[...]
  k_o[...] = _xwt(x0, w4[0:D, :]) + b4[0:1, :]
  qv_o[:, 0:D] = _xwt(x0, w4[D:2 * D, :]) + b4[1:2, :]
  qv_o[:, D:2 * D] = _xwt(x0, w4[2 * D:3 * D, :]) + b4[2:3, :]
  b_o[...] = _xwt(x0, w4[3 * D:4 * D, :]) + b4[3:4, :]


def _tc2_body(base_ref, a0_ref, a1_ref, w4, b4, k_o, qv_o, b_o):
  x1 = jnp.maximum(base_ref[...] + a0_ref[...] + a1_ref[...], 0.0)
  k_o[...] = _xwt(x1, w4[0:D, :]) + b4[0:1, :]
  qv_o[:, 0:D] = _xwt(x1, w4[D:2 * D, :]) + b4[1:2, :]
  qv_o[:, D:2 * D] = _xwt(x1, w4[2 * D:3 * D, :]) + b4[2:3, :]
  b_o[...] = _xwt(x1, w4[3 * D:4 * D, :]) + b4[3:4, :]


def _tc3_body(base_ref, a0_ref, a1_ref, o_ref):
  o_ref[...] = base_ref[...] + a0_ref[...] + a1_ref[...]


_qv_spec = pl.BlockSpec((RB, 2 * D), lambda i: (i, 0))
_node_out = [jax.ShapeDtypeStruct((NPAD, D), jnp.float32),
             jax.ShapeDtypeStruct((NPAD, 2 * D), jnp.float32),
             jax.ShapeDtypeStruct((NPAD, D), jnp.float32)]

_tc1 = pl.pallas_call(
    _tc1_body,
    grid=(GRID,),
    in_specs=[_row_spec, _wh_spec, _bh_spec, _w_spec, _b_spec],
    out_specs=[_row_spec, _qv_spec, _row_spec],
    out_shape=_node_out,
)

_tc2 = pl.pallas_call(
    _tc2_body,
    grid=(GRID,),
    in_specs=[_row_spec, _agg0_spec, _agg1_spec, _w_spec, _b_spec],
    out_specs=[_row_spec, _qv_spec, _row_spec],
    out_shape=_node_out,
)

_tc3 = pl.pallas_call(
    _tc3_body,
    grid=(GRID,),
    in_specs=[_row_spec, _agg0_spec, _agg1_spec],
    out_specs=_row_spec,
    out_shape=jax.ShapeDtypeStruct((NPAD, D), jnp.float32),
)


def _pos_encoding(t):
  tf = t[:, None].astype(jnp.float32)
  inv_freq = 1.0 / (10000.0 ** (jnp.arange(0, D, 2).astype(jnp.float32) / D))
  a = jnp.sin(tf * inv_freq)
  b = jnp.cos(tf * inv_freq)
  pe = jnp.stack([a, b], axis=-1).reshape(t.shape[0], D)
  return pe


def kernel(data, edge_index, t, W_hidden, b_hidden,
           Wk1, bk1, Wq1, bq1, Wv1, bv1, Ws1, bias1,
           Wk2, bk2, Wq2, bq2, Wv2, bv2, Ws2, bias2):
  pe = _pos_encoding(t)
  b_eff = (b_hidden + pe[0]).reshape(1, D)

  x_in = jnp.zeros((NPAD, D), jnp.float32).at[:N].set(data[0])
  epad = jnp.zeros((2, CH), edge_index.dtype)
  eip = jnp.concatenate([edge_index, epad], axis=1)
  src = eip[0]
  dst = eip[1]
  zeros = jnp.zeros((RPT, D), jnp.float32)

  w41 = jnp.concatenate([Wk1, Wq1, Wv1, Ws1], axis=0)
  b41 = jnp.stack([bk1, bq1, bv1, bias1])
  w42 = jnp.concatenate([Wk2, Wq2, Wv2, Ws2], axis=0)
  b42 = jnp.stack([bk2, bq2, bv2, bias2])

  k1, qv1, base1 = _tc1(x_in, W_hidden, b_eff, w41, b41)
  agg1 = _sc_agg(k1, qv1, src, dst, zeros)
  k2, qv2, base2 = _tc2(base1, agg1, agg1, w42, b42)
  agg2 = _sc_agg(k2, qv2, src, dst, zeros)
  out = _tc3(base2, agg2, agg2)
  return out[:N][None]


# speedup vs baseline: 5.5932x; 5.5932x over previous
import functools

import jax
import jax.numpy as jnp
from jax import lax
from jax.experimental import pallas as pl
from jax.experimental.pallas import tpu as pltpu
from jax.experimental.pallas import tpu_sc as plsc

N = 10000
E = 320000
D = 128
NPAD = 10240
NTILES = 32
PER_TILE = E // NTILES
CH = 40
NCH = PER_TILE // CH
RPT = NPAD // 16
UNROLL = 6
STEADY = (NCH - 4) // UNROLL


def _make_sc_agg():
  mesh = plsc.VectorSubcoreMesh(core_axis_name="c", subcore_axis_name="s")

  @functools.partial(
      pl.kernel,
      mesh=mesh,
      out_type=jax.ShapeDtypeStruct((2 * NPAD, D), jnp.float32),
      scratch_types=[
          pltpu.VMEM((CH, D), jnp.float32),
          pltpu.VMEM((CH, D), jnp.float32),
          pltpu.VMEM((CH, D), jnp.float32),
          pltpu.VMEM((CH, D), jnp.float32),
          pltpu.VMEM((CH, D), jnp.float32),
          pltpu.VMEM((CH, D), jnp.float32),
          pltpu.VMEM((CH, D), jnp.float32),
          pltpu.VMEM((CH, D), jnp.float32),
          pltpu.VMEM((CH,), jnp.int32),
          pltpu.VMEM((CH,), jnp.int32),
          pltpu.VMEM((CH,), jnp.int32),
          pltpu.VMEM((CH,), jnp.int32),
          pltpu.VMEM((CH,), jnp.int32),
          pltpu.VMEM((CH,), jnp.int32),
          pltpu.VMEM((CH,), jnp.int32),
          pltpu.VMEM((CH,), jnp.int32),
          pltpu.VMEM_SHARED((NPAD, D), jnp.float32),
          pltpu.SemaphoreType.DMA,
          pltpu.SemaphoreType.DMA,
          pltpu.SemaphoreType.DMA,
          pltpu.SemaphoreType.DMA,
          pltpu.SemaphoreType.DMA,
          pltpu.SemaphoreType.DMA,
          pltpu.SemaphoreType.DMA,
          pltpu.SemaphoreType.DMA,
          pltpu.SemaphoreType.DMA,
      ],
  )
  def sc_agg(k_hbm, q_hbm, v_hbm, src_hbm, dst_hbm, zeros_hbm, out_hbm,
             kb0, qb0, vb0, kb1, qb1, vb1, mb0, mb1,
             is0, id0, is1, id1, is2, id2, pd0, pd1,
             acc, semg0, semg1, semi0, semi1, semi2,
             semsc0, semsc1, sempd0, sempd1):
    c = lax.axis_index("c")
    s = lax.axis_index("s")
    tid = c * 16 + s
    G = ((kb0, qb0, vb0, semg0), (kb1, qb1, vb1, semg1))
    I = ((is0, id0, semi0), (is1, id1, semi1), (is2, id2, semi2))
    M = ((mb0, pd0, semsc0, sempd0), (mb1, pd1, semsc1, sempd1))

    pltpu.sync_copy(zeros_hbm, acc.at[pl.ds(s * RPT, RPT)])
    plsc.subcore_barrier()

    def fire_idx(ci, iu):
      isrc, idst, semi = I[iu]
      base = tid * PER_TILE + ci * CH
      pltpu.async_copy(src_hbm.at[pl.ds(base, CH)], isrc, semi)
      pltpu.async_copy(dst_hbm.at[pl.ds(base, CH)], idst, semi)

    def fire_gather(gu, iu):
      isrc, idst, semi = I[iu]
      kb, qb, vb, semg = G[gu]
      pltpu.make_async_copy(src_hbm.at[pl.ds(0, CH)], isrc, semi).wait()
      pltpu.make_async_copy(dst_hbm.at[pl.ds(0, CH)], idst, semi).wait()
      pltpu.async_copy(k_hbm.at[idst], kb, semg)
      pltpu.async_copy(q_hbm.at[isrc], qb, semg)
      pltpu.async_copy(v_hbm.at[isrc], vb, semg)

    def fire_pds(ci, mu):
      _, pd, _, sempd = M[mu]
      base = tid * PER_TILE + ci * CH
      pltpu.async_copy(dst_hbm.at[pl.ds(base, CH)], pd, sempd)

    def consume(gu, iu, mu, scwait=True):
      isrc, idst, semi = I[iu]
      kb, qb, vb, semg = G[gu]
      mb, pd, semsc, sempd = M[mu]
      pltpu.make_async_copy(k_hbm.at[idst], kb, semg).wait()
      pltpu.make_async_copy(q_hbm.at[isrc], qb, semg).wait()
      pltpu.make_async_copy(v_hbm.at[isrc], vb, semg).wait()
      if scwait:
        pltpu.make_async_copy(mb, acc.at[pd], semsc).wait()
      pltpu.make_async_copy(dst_hbm.at[pl.ds(0, CH)], pd, sempd).wait()

      def row(r2, rc):
        for rr in range(2):
          r = r2 * 2 + rr
          for cc in range(D // 16):
            sl = pl.ds(cc * 16, 16)
            kv = kb[r, sl]
            qv = qb[r, sl]
            vv = vb[r, sl]
            g = 1.0 + jnp.exp(-(kv + qv))
            mb[r, sl] = vv / g
        return rc

      lax.fori_loop(0, CH // 2, row, 0)
      pltpu.async_copy(mb, acc.at[pd], semsc, add=True)

    def body(ci, u, do_gather=True, do_idx=True, do_pds=True, scwait=True):
      if do_gather:
        fire_gather((u + 1) % 2, (u + 1) % 3)
      consume(u % 2, u % 3, u % 2, scwait)
      if do_pds:
        fire_pds(ci + 2, u % 2)
      if do_idx:
        fire_idx(ci + 3, u % 3)

    fire_idx(0, 0)
    fire_idx(1, 1)
    fire_idx(2, 2)
    fire_pds(0, 0)
    fire_pds(1, 1)
    fire_gather(0, 0)

    body(0, 0, scwait=False)
    body(1, 1, scwait=False)

    def steady(j, carry):
      ci = 2 + j * UNROLL
      for u in range(UNROLL):
        body(ci + u, (2 + u) % 6)
      return carry

    lax.fori_loop(0, STEADY, steady, 0)

    body(NCH - 2, (NCH - 2) % 6, do_idx=False, do_pds=False)
    body(NCH - 1, (NCH - 1) % 6, do_gather=False, do_idx=False, do_pds=False)

    mb, pd, semsc, _ = M[(NCH - 2) % 2]
    pltpu.make_async_copy(mb, acc.at[pd], semsc).wait()
    mb, pd, semsc, _ = M[(NCH - 1) % 2]
    pltpu.make_async_copy(mb, acc.at[pd], semsc).wait()
    isrc, idst, semi = I[NCH % 3]
    pltpu.make_async_copy(src_hbm.at[pl.ds(0, CH)], isrc, semi).wait()
    pltpu.make_async_copy(dst_hbm.at[pl.ds(0, CH)], idst, semi).wait()

    plsc.subcore_barrier()
    pltpu.sync_copy(acc.at[pl.ds(s * RPT, RPT)],
                    out_hbm.at[pl.ds(c * NPAD + s * RPT, RPT)])

  return sc_agg


_SC_AGG_CACHE = []


def _sc_agg(*args):
  if not _SC_AGG_CACHE:
    _SC_AGG_CACHE.append(_make_sc_agg())
  return _SC_AGG_CACHE[0](*args)


RB = 640
GRID = NPAD // RB

_row_spec = pl.BlockSpec((RB, D), lambda i: (i, 0))
_agg0_spec = pl.BlockSpec((RB, D), lambda i: (i, 0))
_agg1_spec = pl.BlockSpec((RB, D), lambda i: (GRID + i, 0))
_w_spec = pl.BlockSpec((4 * D, D), lambda i: (0, 0))
_b_spec = pl.BlockSpec((4, D), lambda i: (0, 0))
_wh_spec = pl.BlockSpec((D, D), lambda i: (0, 0))
_bh_spec = pl.BlockSpec((1, D), lambda i: (0, 0))

_DN = (((1,), (1,)), ((), ()))


def _xwt(x, w):
  return lax.dot_general(x, w, _DN, preferred_element_type=jnp.float32)


def _tc1_body(x_ref, wh, beff, w4, b4, k_o, q_o, v_o, b_o):
  x0 = _xwt(x_ref[...], wh[...]) + beff[...]
  k_o[...] = _xwt(x0, w4[0:D, :]) + b4[0:1, :]
  q_o[...] = _xwt(x0, w4[D:2 * D, :]) + b4[1:2, :]
  v_o[...] = _xwt(x0, w4[2 * D:3 * D, :]) + b4[2:3, :]
  b_o[...] = _xwt(x0, w4[3 * D:4 * D, :]) + b4[3:4, :]


def _tc2_body(base_ref, a0_ref, a1_ref, w4, b4, k_o, q_o, v_o, b_o):
  x1 = jnp.maximum(base_ref[...] + a0_ref[...] + a1_ref[...], 0.0)
  k_o[...] = _xwt(x1, w4[0:D, :]) + b4[0:1, :]
  q_o[...] = _xwt(x1, w4[D:2 * D, :]) + b4[1:2, :]
  v_o[...] = _xwt(x1, w4[2 * D:3 * D, :]) + b4[2:3, :]
  b_o[...] = _xwt(x1, w4[3 * D:4 * D, :]) + b4[3:4, :]


def _tc3_body(base_ref, a0_ref, a1_ref, o_ref):
  o_ref[...] = base_ref[...] + a0_ref[...] + a1_ref[...]


_node_out = [jax.ShapeDtypeStruct((NPAD, D), jnp.float32)] * 4

_tc1 = pl.pallas_call(
    _tc1_body,
    grid=(GRID,),
    in_specs=[_row_spec, _wh_spec, _bh_spec, _w_spec, _b_spec],
    out_specs=[_row_spec] * 4,
    out_shape=_node_out,
)

_tc2 = pl.pallas_call(
    _tc2_body,
    grid=(GRID,),
    in_specs=[_row_spec, _agg0_spec, _agg1_spec, _w_spec, _b_spec],
    out_specs=[_row_spec] * 4,
    out_shape=_node_out,
)

_tc3 = pl.pallas_call(
    _tc3_body,
    grid=(GRID,),
    in_specs=[_row_spec, _agg0_spec, _agg1_spec],
    out_specs=_row_spec,
    out_shape=jax.ShapeDtypeStruct((NPAD, D), jnp.float32),
)


def _pos_encoding(t):
  tf = t[:, None].astype(jnp.float32)
  inv_freq = 1.0 / (10000.0 ** (jnp.arange(0, D, 2).astype(jnp.float32) / D))
  a = jnp.sin(tf * inv_freq)
  b = jnp.cos(tf * inv_freq)
  pe = jnp.stack([a, b], axis=-1).reshape(t.shape[0], D)
  return pe


def kernel(data, edge_index, t, W_hidden, b_hidden,
           Wk1, bk1, Wq1, bq1, Wv1, bv1, Ws1, bias1,
           Wk2, bk2, Wq2, bq2, Wv2, bv2, Ws2, bias2):
  pe = _pos_encoding(t)
  b_eff = (b_hidden + pe[0]).reshape(1, D)

  x_in = jnp.zeros((NPAD, D), jnp.float32).at[:N].set(data[0])
  epad = jnp.zeros((2, CH), edge_index.dtype)
  eip = jnp.concatenate([edge_index, epad], axis=1)
  src = eip[0]
  dst = eip[1]
  zeros = jnp.zeros((RPT, D), jnp.float32)

  w41 = jnp.concatenate([Wk1, Wq1, Wv1, Ws1], axis=0)
  b41 = jnp.stack([bk1, bq1, bv1, bias1])
  w42 = jnp.concatenate([Wk2, Wq2, Wv2, Ws2], axis=0)
  b42 = jnp.stack([bk2, bq2, bv2, bias2])

  k1, q1, v1, base1 = _tc1(x_in, W_hidden, b_eff, w41, b41)
  agg1 = _sc_agg(k1, q1, v1, src, dst, zeros)
  k2, q2, v2, base2 = _tc2(base1, agg1, agg1, w42, b42)
  agg2 = _sc_agg(k2, q2, v2, src, dst, zeros)
  out = _tc3(base2, agg2, agg2)
  return out[:N][None]
